# Initial kernel scaffold; baseline (speedup 1.0000x reference)
import jax
import jax.numpy as jnp
from jax.experimental import pallas as pl


def kernel(x, edge_index, edge_index_u, edge_index_v, edge_type, edge_norm, W_u0, b_u0, W_v0, b_v0, W_uv0, b_uv0, W_u1, b_u1, W_v1, b_v1, W_uv1, b_uv1, W_du, b_du, W_di, b_di, basis, coefs):
    raise NotImplementedError("write your pallas kernel here")


# speedup vs baseline: 2.3884x; 2.3884x over previous
import functools

import jax
import jax.numpy as jnp
from jax import lax
from jax.experimental import pallas as pl
from jax.experimental.pallas import tpu as pltpu
from jax.experimental.pallas import tpu_sc as plsc

_N = 4096
_U = 2048
_E = 65536
_E2 = 131072
_R = 5
_NB = 2
_FD = 32

_NC = 2
_NS = 16
_NW = _NC * _NS
_CH = 128


def _segsum_sc(table, src1d, dst1d, zeros, width, n_rows, do_gather=True):
  nedge = src1d.shape[0]
  nchunk = nedge // (_NW * _CH)
  assert nedge % (_NW * _CH) == 0
  assert width in (1, 128)
  nr = n_rows // _NS

  if width == 1:
    ashape = (n_rows,)
    rshape = (_CH,)
  else:
    ashape = (n_rows, width)
    rshape = (_CH, width)
  oshape = (_NC,) + ashape

  trows = table.shape[0]
  tr = trows // _NS
  stage_tbl = do_gather and width == 1
  mesh = plsc.VectorSubcoreMesh(core_axis_name="c", subcore_axis_name="s")

  def body(table_r, src_r, dst_r, zero_r, out_r, srcf, dstf,
           rows, tbl, acc, sem):
    c = lax.axis_index("c")
    s = lax.axis_index("s")
    wid = s * _NC + c
    pltpu.sync_copy(zero_r.at[pl.ds(s * nr, nr)], acc.at[pl.ds(s * nr, nr)])
    if stage_tbl:
      pltpu.sync_copy(table_r.at[pl.ds(s * tr, tr)], tbl.at[pl.ds(s * tr, tr)])
    if not do_gather:
      pltpu.sync_copy(table_r, rows)
    plsc.subcore_barrier()
    base = wid * nchunk * _CH

    def chunk(t, carry):
      pltpu.sync_copy(dst_r.at[pl.ds(base + t * _CH, _CH)], dstf)
      if do_gather:
        pltpu.sync_copy(src_r.at[pl.ds(base + t * _CH, _CH)], srcf)
        src_tbl = tbl if stage_tbl else table_r
        pltpu.async_copy(src_tbl.at[srcf], rows, sem).wait()
      pltpu.sync_copy(rows, acc.at[dstf], add=True)
      return carry

    lax.fori_loop(0, nchunk, chunk, 0)
    plsc.subcore_barrier()
    pltpu.sync_copy(acc.at[pl.ds(s * nr, nr)],
                    out_r.at[c, pl.ds(s * nr, nr)])

  tshape = (trows,) if width == 1 else (trows, width)
  f = pl.kernel(
      body,
      out_type=jax.ShapeDtypeStruct(oshape, jnp.float32),
      mesh=mesh,
      scratch_types=[
          pltpu.VMEM((_CH,), jnp.int32),
          pltpu.VMEM((_CH,), jnp.int32),
          pltpu.VMEM(rshape, jnp.float32),
          pltpu.VMEM_SHARED(tshape if stage_tbl else (8,), jnp.float32),
          pltpu.VMEM_SHARED(ashape, jnp.float32),
          pltpu.SemaphoreType.DMA,
      ],
  )
  return f(table, src1d, dst1d, zeros)


_SC_ON = (True, True, True, True, True)


def _segsum_dbg(table, src1d, dst1d, zeros, width, n_rows, do_gather=True):
  del zeros
  if do_gather:
    vals = table[src1d]
  else:
    vals = jnp.ones((src1d.shape[0],) + (() if width == 1 else (width,)),
                    jnp.float32)
  h = src1d.shape[0] // 2
  p0 = jax.ops.segment_sum(vals[:h], dst1d[:h], num_segments=n_rows)
  p1 = jax.ops.segment_sum(vals[h:], dst1d[h:], num_segments=n_rows)
  return jnp.stack([p0, p1], axis=0)


def _segsum(stage, *args, **kw):
  if _SC_ON[stage]:
    return _segsum_sc(*args, **kw)
  return _segsum_dbg(*args, **kw)


def _dense(body, out_shape, *args):
  return pl.pallas_call(body, out_shape=out_shape)(*args)


def _c0_body(degp_r, x_r, dinv_cat_r, dinv_uv_r, y_r):
  degp = degp_r[...]
  deg = degp[0] + degp[1] + 1.0
  dinv = lax.rsqrt(deg)
  dinv_cat = dinv[:_N]
  dinv_cat_r[...] = dinv_cat
  dinv_uv_r[...] = dinv[_N:]
  y_r[...] = dinv_cat * x_r[...]


def _c1_body(tp_r, y_r, dinv_cat_r, dinv_uv_r, s_r, z2_r):
  tp = tp_r[...]
  dinv_uv = dinv_uv_r[...]
  s = dinv_cat_r[...] * (tp[0] + tp[1] + y_r[...])
  s_r[...] = s
  p = dinv_uv * s
  z = jnp.zeros_like(p)
  mask = lax.broadcasted_iota(jnp.int32, (_N, 1), 0) < _U
  z2u = jnp.concatenate([p, dinv_uv, z, z], axis=1)
  z2v = jnp.concatenate([z, z, p, dinv_uv], axis=1)
  z2_r[...] = jnp.concatenate(
      [jnp.where(mask, z2u, z2v), jnp.zeros((_N, 124), jnp.float32)], axis=1)


def _c2_body(v4p_r, s_r, dinv_cat_r, dinv_uv_r, wu0_r, bu0_r, wv0_r, bv0_r,
             wuv0_r, buv0_r, h1_r, g1_r):
  v4p = v4p_r[...]
  v4 = v4p[0, :, :4] + v4p[1, :, :4]
  dinv_uv = dinv_uv_r[...]
  s = s_r[...]
  mask = lax.broadcasted_iota(jnp.int32, (_N, 1), 0) < _U
  wstack = jnp.concatenate([wu0_r[...], bu0_r[...], wv0_r[...], bv0_r[...]],
                           axis=0)
  wr = jnp.where(mask, wu0_r[...], wv0_r[...])
  br = jnp.where(mask, bu0_r[...], bv0_r[...])
  h = s * wr + br
  agg = dinv_uv * jnp.dot(v4, wstack, preferred_element_type=jnp.float32, precision=lax.Precision.HIGHEST)
  agg = agg + (dinv_uv * dinv_uv) * h
  x1 = jnp.dot(agg, wuv0_r[...], preferred_element_type=jnp.float32, precision=lax.Precision.HIGHEST)
  h1 = jnp.maximum(x1 + buv0_r[...], 0.0)
  h1_r[...] = h1
  g1_r[...] = jnp.concatenate(
      [dinv_cat_r[...] * h1, jnp.zeros((_N, 64), jnp.float32)], axis=1)


def _c3_body(tpp_r, h1_r, dinv_cat_r, dinv_uv_r, wu1_r, bu1_r, wv1_r, bv1_r,
             x2_r, g2_r):
  tpp = tpp_r[...]
  t = tpp[0, :, :64] + tpp[1, :, :64]
  dinv_cat = dinv_cat_r[...]
  h1 = h1_r[...]
  agg2 = dinv_cat * t + (dinv_cat * dinv_cat) * h1
  xu = jnp.dot(agg2, wu1_r[...], preferred_element_type=jnp.float32, precision=lax.Precision.HIGHEST) + bu1_r[...]
  xv = jnp.dot(agg2, wv1_r[...], preferred_element_type=jnp.float32, precision=lax.Precision.HIGHEST) + bv1_r[...]
  mask = lax.broadcasted_iota(jnp.int32, (_N, 1), 0) < _U
  x2 = jnp.where(mask, xu, xv)
  x2_r[...] = x2
  g2_r[...] = jnp.concatenate(
      [dinv_uv_r[...] * x2, jnp.zeros((_N, 64), jnp.float32)], axis=1)


def _c4_body(t2p_r, x2_r, dinv_uv_r, wuv1_r, buv1_r, wdu_r, bdu_r, wdi_r,
             bdi_r, basis3_r, coefs_r, ufeat_r, k5_r):
  t2p = t2p_r[...]
  t2 = t2p[0, :, :64] + t2p[1, :, :64]
  dinv_uv = dinv_uv_r[...]
  x2 = x2_r[...]
  agg3 = dinv_uv * t2 + (dinv_uv * dinv_uv) * x2
  x3 = jnp.dot(agg3, wuv1_r[...], preferred_element_type=jnp.float32, precision=lax.Precision.HIGHEST) + buv1_r[...]
  u_feat = jnp.dot(x3[:_U], wdu_r[...],
                   preferred_element_type=jnp.float32, precision=lax.Precision.HIGHEST) + bdu_r[...]
  i_feat = jnp.dot(x3[_U:], wdi_r[...],
                   preferred_element_type=jnp.float32, precision=lax.Precision.HIGHEST) + bdi_r[...]
  ufeat_r[...] = u_feat
  basis3 = basis3_r[...]
  coefs = coefs_r[...]
  for r in range(_R):
    q = coefs[r, 0] * basis3[0] + coefs[r, 1] * basis3[1]
    k5_r[r] = lax.dot_general(q, i_feat, (((1,), (1,)), ((), ())),
                              preferred_element_type=jnp.float32, precision=lax.Precision.HIGHEST)


def _c5_body(u_r, qi_r, out_r):
  out_r[...] = jnp.dot(u_r[...], qi_r[...],
                       preferred_element_type=jnp.float32, precision=lax.Precision.HIGHEST)


def kernel(x, edge_index, edge_index_u, edge_index_v, edge_type, edge_norm,
           W_u0, b_u0, W_v0, b_v0, W_uv0, b_uv0,
           W_u1, b_u1, W_v1, b_v1, W_uv1, b_uv1,
           W_du, b_du, W_di, b_di, basis, coefs):
  del edge_type, edge_norm

  f32 = jnp.float32
  ei = edge_index.astype(jnp.int32)
  ei_u = edge_index_u.astype(jnp.int32)
  ei_v = edge_index_v.astype(jnp.int32)

  src_cat = jnp.concatenate([ei_u[0], ei_v[0]])
  dst_cat = jnp.concatenate([ei_u[1], ei_v[1]])
  src_uv = ei[0]
  dst_uv = ei[1]

  src_all = jnp.concatenate([src_cat, src_uv])
  dst_all = jnp.concatenate([dst_cat, dst_uv + _N])
  ones_rows = jnp.ones((_CH,), f32)
  z2n = jnp.zeros((2 * _N,), f32)
  degp = _segsum(0, ones_rows, src_all, dst_all, z2n, 1, 2 * _N,
                    do_gather=False)

  xc = x.reshape(_N, 1).astype(f32)
  dinv_cat, dinv_uv, y = _dense(
      _c0_body,
      [jax.ShapeDtypeStruct((_N, 1), f32)] * 3,
      degp.reshape(_NC, 2 * _N, 1), xc)

  zn = jnp.zeros((_N,), f32)
  tp = _segsum(1, y.reshape(_N), src_cat, dst_cat, zn, 1, _N)

  s, z2t = _dense(
      _c1_body,
      [jax.ShapeDtypeStruct((_N, 1), f32), jax.ShapeDtypeStruct((_N, 128), f32)],
      tp.reshape(_NC, _N, 1), y, dinv_cat, dinv_uv)

  zn128 = jnp.zeros((_N, 128), f32)
  v4p = _segsum(2, z2t, src_uv, dst_uv, zn128, 128, _N)

  b1r = lambda v: v.reshape(1, -1)
  h1, g1 = _dense(
      _c2_body,
      [jax.ShapeDtypeStruct((_N, 64), f32), jax.ShapeDtypeStruct((_N, 128), f32)],
      v4p, s, dinv_cat, dinv_uv,
      W_u0, b1r(b_u0), W_v0, b1r(b_v0), W_uv0, b1r(b_uv0))

  tpp = _segsum(3, g1, src_cat, dst_cat, zn128, 128, _N)

  x2, g2 = _dense(
      _c3_body,
      [jax.ShapeDtypeStruct((_N, 64), f32), jax.ShapeDtypeStruct((_N, 128), f32)],
      tpp, h1, dinv_cat, dinv_uv, W_u1, b1r(b_u1), W_v1, b1r(b_v1))

  t2p = _segsum(4, g2, src_uv, dst_uv, zn128, 128, _N)

  ufeat, k5 = _dense(
      _c4_body,
      [jax.ShapeDtypeStruct((_U, _FD), f32),
       jax.ShapeDtypeStruct((_R, _FD, _N - _U), f32)],
      t2p, x2, dinv_uv, W_uv1, b1r(b_uv1), W_du, b1r(b_du), W_di, b1r(b_di),
      basis.reshape(_NB, _FD, _FD), coefs)

  qi = jnp.transpose(k5, (1, 2, 0)).reshape(_FD, (_N - _U) * _R)

  ncols = (_N - _U) * _R
  bn = 1280
  out2d = pl.pallas_call(
      _c5_body,
      grid=(ncols // bn,),
      in_specs=[
          pl.BlockSpec((_U, _FD), lambda j: (0, 0)),
          pl.BlockSpec((_FD, bn), lambda j: (0, j)),
      ],
      out_specs=pl.BlockSpec((_U, bn), lambda j: (0, j)),
      out_shape=jax.ShapeDtypeStruct((_U, ncols), f32),
  )(ufeat, qi)

  return out2d.reshape(_U * (_N - _U), _R)


# speedup vs baseline: 2.4943x; 1.0444x over previous
import functools

import jax
import jax.numpy as jnp
from jax import lax
from jax.experimental import pallas as pl
from jax.experimental.pallas import tpu as pltpu
from jax.experimental.pallas import tpu_sc as plsc

_N = 4096
_U = 2048
_E = 65536
_E2 = 131072
_R = 5
_NB = 2
_FD = 32

_NC = 2
_NS = 16
_NW = _NC * _NS
_CH = 128


def _segsum_sc(table, src1d, dst1d, zeros, width, n_rows, do_gather=True):
  nedge = src1d.shape[0]
  nchunk = nedge // (_NW * _CH)
  assert nedge % (_NW * _CH) == 0
  assert width in (1, 128)
  nr = n_rows // _NS

  if width == 1:
    ashape = (n_rows,)
    rshape = (_CH,)
  else:
    ashape = (n_rows, width)
    rshape = (_CH, width)
  oshape = (_NC,) + ashape

  trows = table.shape[0]
  tr = trows // _NS
  stage_tbl = do_gather and width == 1
  mesh = plsc.VectorSubcoreMesh(core_axis_name="c", subcore_axis_name="s")

  def body(table_r, src_r, dst_r, zero_r, out_r, srcf0, srcf1, dstf0, dstf1,
           rows0, rows1, tbl, acc, semi0, semi1, semg0, semg1):
    srcf = (srcf0, srcf1)
    dstf = (dstf0, dstf1)
    rows = (rows0, rows1)
    semi = (semi0, semi1)
    semg = (semg0, semg1)
    c = lax.axis_index("c")
    s = lax.axis_index("s")
    wid = s * _NC + c
    pltpu.sync_copy(zero_r.at[pl.ds(s * nr, nr)], acc.at[pl.ds(s * nr, nr)])
    if stage_tbl:
      pltpu.sync_copy(table_r.at[pl.ds(s * tr, tr)], tbl.at[pl.ds(s * tr, tr)])
    if not do_gather:
      pltpu.sync_copy(table_r, rows0)
    plsc.subcore_barrier()
    base = wid * nchunk * _CH
    src_tbl = tbl if stage_tbl else table_r
    last = nedge - _CH

    def off(t):
      return jnp.minimum(base + t * _CH, last)

    def wait_idx(b, want_src):
      if want_src:
        pltpu.make_async_copy(src_r.at[pl.ds(0, _CH)], srcf[b], semi[b]).wait()
      pltpu.make_async_copy(dst_r.at[pl.ds(0, _CH)], dstf[b], semi[b]).wait()

    def wait_gather(b):
      pltpu.make_async_copy(table_r.at[pl.ds(0, rshape[0])]
                            if width == 1 else
                            table_r.at[pl.ds(0, _CH), :],
                            rows[b], semg[b]).wait()

    if do_gather:
      pltpu.async_copy(src_r.at[pl.ds(off(0), _CH)], srcf[0], semi[0])
      pltpu.async_copy(dst_r.at[pl.ds(off(0), _CH)], dstf[0], semi[0])
      pltpu.async_copy(src_r.at[pl.ds(off(1), _CH)], srcf[1], semi[1])
      pltpu.async_copy(dst_r.at[pl.ds(off(1), _CH)], dstf[1], semi[1])
      wait_idx(0, True)
      pltpu.async_copy(src_tbl.at[srcf[0]], rows[0], semg[0])

      def phase(t, b):
        b2 = 1 - b
        wait_idx(b2, True)
        pltpu.async_copy(src_tbl.at[srcf[b2]], rows[b2], semg[b2])
        wait_gather(b)
        pltpu.async_copy(src_r.at[pl.ds(off(t + 2), _CH)], srcf[b], semi[b])
        pltpu.sync_copy(rows[b], acc.at[dstf[b]], add=True)
        pltpu.async_copy(dst_r.at[pl.ds(off(t + 2), _CH)], dstf[b], semi[b])

      def pair(j, carry):
        phase(2 * j, 0)
        phase(2 * j + 1, 1)
        return carry

      lax.fori_loop(0, nchunk // 2, pair, 0)
      wait_gather(0)
      wait_idx(1, True)
    else:
      pltpu.async_copy(dst_r.at[pl.ds(off(0), _CH)], dstf[0], semi[0])
      pltpu.async_copy(dst_r.at[pl.ds(off(1), _CH)], dstf[1], semi[1])
      wait_idx(0, False)

      def phase0(t, b):
        b2 = 1 - b
        wait_idx(b2, False)
        pltpu.sync_copy(rows0, acc.at[dstf[b]], add=True)
        pltpu.async_copy(dst_r.at[pl.ds(off(t + 2), _CH)], dstf[b], semi[b])

      def pair0(j, carry):
        phase0(2 * j, 0)
        phase0(2 * j + 1, 1)
        return carry

      lax.fori_loop(0, nchunk // 2, pair0, 0)
      wait_idx(1, False)

    plsc.subcore_barrier()
    pltpu.sync_copy(acc.at[pl.ds(s * nr, nr)],
                    out_r.at[c, pl.ds(s * nr, nr)])

  tshape = (trows,) if width == 1 else (trows, width)
  f = pl.kernel(
      body,
      out_type=jax.ShapeDtypeStruct(oshape, jnp.float32),
      mesh=mesh,
      scratch_types=[
          pltpu.VMEM((_CH,), jnp.int32),
          pltpu.VMEM((_CH,), jnp.int32),
          pltpu.VMEM((_CH,), jnp.int32),
          pltpu.VMEM((_CH,), jnp.int32),
          pltpu.VMEM(rshape, jnp.float32),
          pltpu.VMEM(rshape, jnp.float32),
          pltpu.VMEM_SHARED(tshape if stage_tbl else (8,), jnp.float32),
          pltpu.VMEM_SHARED(ashape, jnp.float32),
          pltpu.SemaphoreType.DMA,
          pltpu.SemaphoreType.DMA,
          pltpu.SemaphoreType.DMA,
          pltpu.SemaphoreType.DMA,
      ],
  )
  return f(table, src1d, dst1d, zeros)


_SC_ON = (True, True, True, True, True)


def _segsum_dbg(table, src1d, dst1d, zeros, width, n_rows, do_gather=True):
  del zeros
  if do_gather:
    vals = table[src1d]
  else:
    vals = jnp.ones((src1d.shape[0],) + (() if width == 1 else (width,)),
                    jnp.float32)
  h = src1d.shape[0] // 2
  p0 = jax.ops.segment_sum(vals[:h], dst1d[:h], num_segments=n_rows)
  p1 = jax.ops.segment_sum(vals[h:], dst1d[h:], num_segments=n_rows)
  return jnp.stack([p0, p1], axis=0)


def _segsum(stage, *args, **kw):
  if _SC_ON[stage]:
    return _segsum_sc(*args, **kw)
  return _segsum_dbg(*args, **kw)


def _dense(body, out_shape, *args):
  return pl.pallas_call(body, out_shape=out_shape)(*args)


def _c0_body(degp_r, x_r, dinv_cat_r, dinv_uv_r, y_r):
  degp = degp_r[...]
  deg = degp[0] + degp[1] + 1.0
  dinv = lax.rsqrt(deg)
  dinv_cat = dinv[:_N]
  dinv_cat_r[...] = dinv_cat
  dinv_uv_r[...] = dinv[_N:]
  y_r[...] = dinv_cat * x_r[...]


def _c1_body(tp_r, y_r, dinv_cat_r, dinv_uv_r, s_r, z2_r):
  tp = tp_r[...]
  dinv_uv = dinv_uv_r[...]
  s = dinv_cat_r[...] * (tp[0] + tp[1] + y_r[...])
  s_r[...] = s
  p = dinv_uv * s
  z = jnp.zeros_like(p)
  mask = lax.broadcasted_iota(jnp.int32, (_N, 1), 0) < _U
  z2u = jnp.concatenate([p, dinv_uv, z, z], axis=1)
  z2v = jnp.concatenate([z, z, p, dinv_uv], axis=1)
  z2_r[...] = jnp.concatenate(
      [jnp.where(mask, z2u, z2v), jnp.zeros((_N, 124), jnp.float32)], axis=1)


def _c2_body(v4p_r, s_r, dinv_cat_r, dinv_uv_r, wu0_r, bu0_r, wv0_r, bv0_r,
             wuv0_r, buv0_r, h1_r, g1_r):
  v4p = v4p_r[...]
  v4 = v4p[0, :, :4] + v4p[1, :, :4]
  dinv_uv = dinv_uv_r[...]
  s = s_r[...]
  mask = lax.broadcasted_iota(jnp.int32, (_N, 1), 0) < _U
  wstack = jnp.concatenate([wu0_r[...], bu0_r[...], wv0_r[...], bv0_r[...]],
                           axis=0)
  wr = jnp.where(mask, wu0_r[...], wv0_r[...])
  br = jnp.where(mask, bu0_r[...], bv0_r[...])
  h = s * wr + br
  agg = dinv_uv * jnp.dot(v4, wstack, preferred_element_type=jnp.float32, precision=lax.Precision.HIGHEST)
  agg = agg + (dinv_uv * dinv_uv) * h
  x1 = jnp.dot(agg, wuv0_r[...], preferred_element_type=jnp.float32, precision=lax.Precision.HIGHEST)
  h1 = jnp.maximum(x1 + buv0_r[...], 0.0)
  h1_r[...] = h1
  g1_r[...] = jnp.concatenate(
      [dinv_cat_r[...] * h1, jnp.zeros((_N, 64), jnp.float32)], axis=1)


def _c3_body(tpp_r, h1_r, dinv_cat_r, dinv_uv_r, wu1_r, bu1_r, wv1_r, bv1_r,
             x2_r, g2_r):
  tpp = tpp_r[...]
  t = tpp[0, :, :64] + tpp[1, :, :64]
  dinv_cat = dinv_cat_r[...]
  h1 = h1_r[...]
  agg2 = dinv_cat * t + (dinv_cat * dinv_cat) * h1
  xu = jnp.dot(agg2, wu1_r[...], preferred_element_type=jnp.float32, precision=lax.Precision.HIGHEST) + bu1_r[...]
  xv = jnp.dot(agg2, wv1_r[...], preferred_element_type=jnp.float32, precision=lax.Precision.HIGHEST) + bv1_r[...]
  mask = lax.broadcasted_iota(jnp.int32, (_N, 1), 0) < _U
  x2 = jnp.where(mask, xu, xv)
  x2_r[...] = x2
  g2_r[...] = jnp.concatenate(
      [dinv_uv_r[...] * x2, jnp.zeros((_N, 64), jnp.float32)], axis=1)


def _c4_body(t2p_r, x2_r, dinv_uv_r, wuv1_r, buv1_r, wdu_r, bdu_r, wdi_r,
             bdi_r, basis3_r, coefs_r, ufeat_r, k5_r):
  t2p = t2p_r[...]
  t2 = t2p[0, :, :64] + t2p[1, :, :64]
  dinv_uv = dinv_uv_r[...]
  x2 = x2_r[...]
  agg3 = dinv_uv * t2 + (dinv_uv * dinv_uv) * x2
  x3 = jnp.dot(agg3, wuv1_r[...], preferred_element_type=jnp.float32, precision=lax.Precision.HIGHEST) + buv1_r[...]
  u_feat = jnp.dot(x3[:_U], wdu_r[...],
                   preferred_element_type=jnp.float32, precision=lax.Precision.HIGHEST) + bdu_r[...]
  i_feat = jnp.dot(x3[_U:], wdi_r[...],
                   preferred_element_type=jnp.float32, precision=lax.Precision.HIGHEST) + bdi_r[...]
  ufeat_r[...] = u_feat
  basis3 = basis3_r[...]
  coefs = coefs_r[...]
  for r in range(_R):
    q = coefs[r, 0] * basis3[0] + coefs[r, 1] * basis3[1]
    k5_r[r] = lax.dot_general(q, i_feat, (((1,), (1,)), ((), ())),
                              preferred_element_type=jnp.float32, precision=lax.Precision.HIGHEST)


def _c5_body(u_r, qi_r, out_r):
  out_r[...] = jnp.dot(u_r[...], qi_r[...],
                       preferred_element_type=jnp.float32, precision=lax.Precision.HIGHEST)


def kernel(x, edge_index, edge_index_u, edge_index_v, edge_type, edge_norm,
           W_u0, b_u0, W_v0, b_v0, W_uv0, b_uv0,
           W_u1, b_u1, W_v1, b_v1, W_uv1, b_uv1,
           W_du, b_du, W_di, b_di, basis, coefs):
  del edge_type, edge_norm

  f32 = jnp.float32
  ei = edge_index.astype(jnp.int32)
  ei_u = edge_index_u.astype(jnp.int32)
  ei_v = edge_index_v.astype(jnp.int32)

  src_cat = jnp.concatenate([ei_u[0], ei_v[0]])
  dst_cat = jnp.concatenate([ei_u[1], ei_v[1]])
  src_uv = ei[0]
  dst_uv = ei[1]

  src_all = jnp.concatenate([src_cat, src_uv])
  dst_all = jnp.concatenate([dst_cat, dst_uv + _N])
  ones_rows = jnp.ones((_CH,), f32)
  z2n = jnp.zeros((2 * _N,), f32)
  degp = _segsum(0, ones_rows, src_all, dst_all, z2n, 1, 2 * _N,
                    do_gather=False)

  xc = x.reshape(_N, 1).astype(f32)
  dinv_cat, dinv_uv, y = _dense(
      _c0_body,
      [jax.ShapeDtypeStruct((_N, 1), f32)] * 3,
      degp.reshape(_NC, 2 * _N, 1), xc)

  zn = jnp.zeros((_N,), f32)
  tp = _segsum(1, y.reshape(_N), src_cat, dst_cat, zn, 1, _N)

  s, z2t = _dense(
      _c1_body,
      [jax.ShapeDtypeStruct((_N, 1), f32), jax.ShapeDtypeStruct((_N, 128), f32)],
      tp.reshape(_NC, _N, 1), y, dinv_cat, dinv_uv)

  zn128 = jnp.zeros((_N, 128), f32)
  v4p = _segsum(2, z2t, src_uv, dst_uv, zn128, 128, _N)

  b1r = lambda v: v.reshape(1, -1)
  h1, g1 = _dense(
      _c2_body,
      [jax.ShapeDtypeStruct((_N, 64), f32), jax.ShapeDtypeStruct((_N, 128), f32)],
      v4p, s, dinv_cat, dinv_uv,
      W_u0, b1r(b_u0), W_v0, b1r(b_v0), W_uv0, b1r(b_uv0))

  tpp = _segsum(3, g1, src_cat, dst_cat, zn128, 128, _N)

  x2, g2 = _dense(
      _c3_body,
      [jax.ShapeDtypeStruct((_N, 64), f32), jax.ShapeDtypeStruct((_N, 128), f32)],
      tpp, h1, dinv_cat, dinv_uv, W_u1, b1r(b_u1), W_v1, b1r(b_v1))

  t2p = _segsum(4, g2, src_uv, dst_uv, zn128, 128, _N)

  ufeat, k5 = _dense(
      _c4_body,
      [jax.ShapeDtypeStruct((_U, _FD), f32),
       jax.ShapeDtypeStruct((_R, _FD, _N - _U), f32)],
      t2p, x2, dinv_uv, W_uv1, b1r(b_uv1), W_du, b1r(b_du), W_di, b1r(b_di),
      basis.reshape(_NB, _FD, _FD), coefs)

  qi = jnp.transpose(k5, (1, 2, 0)).reshape(_FD, (_N - _U) * _R)

  ncols = (_N - _U) * _R
  bn = 1280
  out2d = pl.pallas_call(
      _c5_body,
      grid=(ncols // bn,),
      in_specs=[
          pl.BlockSpec((_U, _FD), lambda j: (0, 0)),
          pl.BlockSpec((_FD, bn), lambda j: (0, j)),
      ],
      out_specs=pl.BlockSpec((_U, bn), lambda j: (0, j)),
      out_shape=jax.ShapeDtypeStruct((_U, ncols), f32),
  )(ufeat, qi)

  return out2d.reshape(_U * (_N - _U), _R)


# speedup vs baseline: 2.4953x; 1.0004x over previous
import jax
import jax.numpy as jnp
from jax import lax
from jax.experimental import pallas as pl
from jax.experimental.pallas import tpu as pltpu
from jax.experimental.pallas import tpu_sc as plsc

_N = 4096
_U = 2048
_E = 65536
_E2 = 131072
_R = 5
_NB = 2
_FD = 32

_NC = 2
_NS = 16
_NW = _NC * _NS
_CH = 128


def _segsum_sc(table, src1d, dst1d, zeros, width, n_rows, do_gather=True):
  nedge = src1d.shape[0]
  nchunk = nedge // (_NW * _CH)
  assert nedge % (_NW * _CH) == 0
  assert width in (1, 128)
  nr = n_rows // _NS

  if width == 1:
    ashape = (n_rows,)
    rshape = (_CH,)
  else:
    ashape = (n_rows, width)
    rshape = (_CH, width)
  oshape = (_NC,) + ashape

  trows = table.shape[0]
  tr = trows // _NS
  stage_tbl = do_gather and width == 1
  mesh = plsc.VectorSubcoreMesh(core_axis_name="c", subcore_axis_name="s")

  def body(table_r, src_r, dst_r, zero_r, out_r, srcf0, srcf1, dstf0, dstf1,
           rows0, rows1, tbl, acc, semi0, semi1, semg0, semg1):
    srcf = (srcf0, srcf1)
    dstf = (dstf0, dstf1)
    rows = (rows0, rows1)
    semi = (semi0, semi1)
    semg = (semg0, semg1)
    c = lax.axis_index("c")
    s = lax.axis_index("s")
    wid = s * _NC + c
    pltpu.sync_copy(zero_r.at[pl.ds(s * nr, nr)], acc.at[pl.ds(s * nr, nr)])
    if stage_tbl:
      pltpu.sync_copy(table_r.at[pl.ds(s * tr, tr)], tbl.at[pl.ds(s * tr, tr)])
    if not do_gather:
      pltpu.sync_copy(table_r, rows0)
    plsc.subcore_barrier()
    base = wid * nchunk * _CH
    src_tbl = tbl if stage_tbl else table_r
    last = nedge - _CH

    def off(t):
      return jnp.minimum(base + t * _CH, last)

    def wait_idx(b, want_src):
      if want_src:
        pltpu.make_async_copy(src_r.at[pl.ds(0, _CH)], srcf[b], semi[b]).wait()
      pltpu.make_async_copy(dst_r.at[pl.ds(0, _CH)], dstf[b], semi[b]).wait()

    def wait_gather(b):
      pltpu.make_async_copy(table_r.at[pl.ds(0, rshape[0])]
                            if width == 1 else
                            table_r.at[pl.ds(0, _CH), :],
                            rows[b], semg[b]).wait()

    if do_gather:
      pltpu.async_copy(src_r.at[pl.ds(off(0), _CH)], srcf[0], semi[0])
      pltpu.async_copy(dst_r.at[pl.ds(off(0), _CH)], dstf[0], semi[0])
      pltpu.async_copy(src_r.at[pl.ds(off(1), _CH)], srcf[1], semi[1])
      pltpu.async_copy(dst_r.at[pl.ds(off(1), _CH)], dstf[1], semi[1])
      wait_idx(0, True)
      pltpu.async_copy(src_tbl.at[srcf[0]], rows[0], semg[0])

      def phase(t, b):
        b2 = 1 - b
        wait_idx(b2, True)
        pltpu.async_copy(src_tbl.at[srcf[b2]], rows[b2], semg[b2])
        wait_gather(b)
        pltpu.async_copy(src_r.at[pl.ds(off(t + 2), _CH)], srcf[b], semi[b])
        pltpu.sync_copy(rows[b], acc.at[dstf[b]], add=True)
        pltpu.async_copy(dst_r.at[pl.ds(off(t + 2), _CH)], dstf[b], semi[b])

      def pair(j, carry):
        phase(2 * j, 0)
        phase(2 * j + 1, 1)
        return carry

      lax.fori_loop(0, nchunk // 2, pair, 0)
      wait_gather(0)
      wait_idx(1, True)
    else:
      pltpu.async_copy(dst_r.at[pl.ds(off(0), _CH)], dstf[0], semi[0])
      pltpu.async_copy(dst_r.at[pl.ds(off(1), _CH)], dstf[1], semi[1])
      wait_idx(0, False)

      def phase0(t, b):
        b2 = 1 - b
        wait_idx(b2, False)
        pltpu.sync_copy(rows0, acc.at[dstf[b]], add=True)
        pltpu.async_copy(dst_r.at[pl.ds(off(t + 2), _CH)], dstf[b], semi[b])

      def pair0(j, carry):
        phase0(2 * j, 0)
        phase0(2 * j + 1, 1)
        return carry

      lax.fori_loop(0, nchunk // 2, pair0, 0)
      wait_idx(1, False)

    plsc.subcore_barrier()
    pltpu.sync_copy(acc.at[pl.ds(s * nr, nr)],
                    out_r.at[c, pl.ds(s * nr, nr)])

  tshape = (trows,) if width == 1 else (trows, width)
  f = pl.kernel(
      body,
      out_type=jax.ShapeDtypeStruct(oshape, jnp.float32),
      mesh=mesh,
      scratch_types=[
          pltpu.VMEM((_CH,), jnp.int32),
          pltpu.VMEM((_CH,), jnp.int32),
          pltpu.VMEM((_CH,), jnp.int32),
          pltpu.VMEM((_CH,), jnp.int32),
          pltpu.VMEM(rshape, jnp.float32),
          pltpu.VMEM(rshape, jnp.float32),
          pltpu.VMEM_SHARED(tshape if stage_tbl else (8,), jnp.float32),
          pltpu.VMEM_SHARED(ashape, jnp.float32),
          pltpu.SemaphoreType.DMA,
          pltpu.SemaphoreType.DMA,
          pltpu.SemaphoreType.DMA,
          pltpu.SemaphoreType.DMA,
      ],
  )
  return f(table, src1d, dst1d, zeros)


def _dense(body, out_shape, *args):
  return pl.pallas_call(body, out_shape=out_shape)(*args)


def _c0_body(degp_r, x_r, dinv_cat_r, dinv_uv_r, y_r):
  degp = degp_r[...]
  deg = degp[0] + degp[1] + 1.0
  dinv = lax.rsqrt(deg)
  dinv_cat = dinv[:_N]
  dinv_cat_r[...] = dinv_cat
  dinv_uv_r[...] = dinv[_N:]
  y_r[...] = dinv_cat * x_r[...]


def _c1_body(tp_r, y_r, dinv_cat_r, dinv_uv_r, s_r, z2_r):
  tp = tp_r[...]
  dinv_uv = dinv_uv_r[...]
  s = dinv_cat_r[...] * (tp[0] + tp[1] + y_r[...])
  s_r[...] = s
  p = dinv_uv * s
  z = jnp.zeros_like(p)
  mask = lax.broadcasted_iota(jnp.int32, (_N, 1), 0) < _U
  z2u = jnp.concatenate([p, dinv_uv, z, z], axis=1)
  z2v = jnp.concatenate([z, z, p, dinv_uv], axis=1)
  z2_r[...] = jnp.concatenate(
      [jnp.where(mask, z2u, z2v), jnp.zeros((_N, 124), jnp.float32)], axis=1)


def _c2_body(v4p_r, s_r, dinv_cat_r, dinv_uv_r, wu0_r, bu0_r, wv0_r, bv0_r,
             wuv0_r, buv0_r, h1_r, g1_r):
  v4p = v4p_r[...]
  v4 = v4p[0, :, :4] + v4p[1, :, :4]
  dinv_uv = dinv_uv_r[...]
  s = s_r[...]
  mask = lax.broadcasted_iota(jnp.int32, (_N, 1), 0) < _U
  wstack = jnp.concatenate([wu0_r[...], bu0_r[...], wv0_r[...], bv0_r[...]],
                           axis=0)
  wr = jnp.where(mask, wu0_r[...], wv0_r[...])
  br = jnp.where(mask, bu0_r[...], bv0_r[...])
  h = s * wr + br
  agg = dinv_uv * jnp.dot(v4, wstack, preferred_element_type=jnp.float32, precision=lax.Precision.HIGHEST)
  agg = agg + (dinv_uv * dinv_uv) * h
  x1 = jnp.dot(agg, wuv0_r[...], preferred_element_type=jnp.float32, precision=lax.Precision.HIGHEST)
  h1 = jnp.maximum(x1 + buv0_r[...], 0.0)
  h1_r[...] = h1
  g1_r[...] = jnp.concatenate(
      [dinv_cat_r[...] * h1, jnp.zeros((_N, 64), jnp.float32)], axis=1)


def _c3_body(tpp_r, h1_r, dinv_cat_r, dinv_uv_r, wu1_r, bu1_r, wv1_r, bv1_r,
             x2_r, g2_r):
  tpp = tpp_r[...]
  t = tpp[0, :, :64] + tpp[1, :, :64]
  dinv_cat = dinv_cat_r[...]
  h1 = h1_r[...]
  agg2 = dinv_cat * t + (dinv_cat * dinv_cat) * h1
  xu = jnp.dot(agg2, wu1_r[...], preferred_element_type=jnp.float32, precision=lax.Precision.HIGHEST) + bu1_r[...]
  xv = jnp.dot(agg2, wv1_r[...], preferred_element_type=jnp.float32, precision=lax.Precision.HIGHEST) + bv1_r[...]
  mask = lax.broadcasted_iota(jnp.int32, (_N, 1), 0) < _U
  x2 = jnp.where(mask, xu, xv)
  x2_r[...] = x2
  g2_r[...] = jnp.concatenate(
      [dinv_uv_r[...] * x2, jnp.zeros((_N, 64), jnp.float32)], axis=1)


def _c4_body(t2p_r, x2_r, dinv_uv_r, wuv1_r, buv1_r, wdu_r, bdu_r, wdi_r,
             bdi_r, basis3_r, coefs_r, ufeat_r, k5_r):
  t2p = t2p_r[...]
  t2 = t2p[0, :, :64] + t2p[1, :, :64]
  dinv_uv = dinv_uv_r[...]
  x2 = x2_r[...]
  agg3 = dinv_uv * t2 + (dinv_uv * dinv_uv) * x2
  x3 = jnp.dot(agg3, wuv1_r[...], preferred_element_type=jnp.float32, precision=lax.Precision.HIGHEST) + buv1_r[...]
  u_feat = jnp.dot(x3[:_U], wdu_r[...],
                   preferred_element_type=jnp.float32, precision=lax.Precision.HIGHEST) + bdu_r[...]
  i_feat = jnp.dot(x3[_U:], wdi_r[...],
                   preferred_element_type=jnp.float32, precision=lax.Precision.HIGHEST) + bdi_r[...]
  ufeat_r[...] = u_feat
  basis3 = basis3_r[...]
  coefs = coefs_r[...]
  for r in range(_R):
    q = coefs[r, 0] * basis3[0] + coefs[r, 1] * basis3[1]
    k5_r[r] = lax.dot_general(q, i_feat, (((1,), (1,)), ((), ())),
                              preferred_element_type=jnp.float32, precision=lax.Precision.HIGHEST)


def _c5_body(u_r, qi_r, out_r):
  out_r[...] = jnp.dot(u_r[...], qi_r[...],
                       preferred_element_type=jnp.float32, precision=lax.Precision.HIGHEST)


def kernel(x, edge_index, edge_index_u, edge_index_v, edge_type, edge_norm,
           W_u0, b_u0, W_v0, b_v0, W_uv0, b_uv0,
           W_u1, b_u1, W_v1, b_v1, W_uv1, b_uv1,
           W_du, b_du, W_di, b_di, basis, coefs):
  del edge_type, edge_norm

  f32 = jnp.float32
  ei = edge_index.astype(jnp.int32)
  ei_u = edge_index_u.astype(jnp.int32)
  ei_v = edge_index_v.astype(jnp.int32)

  src_cat = jnp.concatenate([ei_u[0], ei_v[0]])
  dst_cat = jnp.concatenate([ei_u[1], ei_v[1]])
  src_uv = ei[0]
  dst_uv = ei[1]

  src_all = jnp.concatenate([src_cat, src_uv])
  dst_all = jnp.concatenate([dst_cat, dst_uv + _N])
  ones_rows = jnp.ones((_CH,), f32)
  z2n = jnp.zeros((2 * _N,), f32)
  degp = _segsum_sc(ones_rows, src_all, dst_all, z2n, 1, 2 * _N,
                    do_gather=False)

  xc = x.reshape(_N, 1).astype(f32)
  dinv_cat, dinv_uv, y = _dense(
      _c0_body,
      [jax.ShapeDtypeStruct((_N, 1), f32)] * 3,
      degp.reshape(_NC, 2 * _N, 1), xc)

  zn = jnp.zeros((_N,), f32)
  tp = _segsum_sc(y.reshape(_N), src_cat, dst_cat, zn, 1, _N)

  s, z2t = _dense(
      _c1_body,
      [jax.ShapeDtypeStruct((_N, 1), f32), jax.ShapeDtypeStruct((_N, 128), f32)],
      tp.reshape(_NC, _N, 1), y, dinv_cat, dinv_uv)

  zn128 = jnp.zeros((_N, 128), f32)
  v4p = _segsum_sc(z2t, src_uv, dst_uv, zn128, 128, _N)

  b1r = lambda v: v.reshape(1, -1)
  h1, g1 = _dense(
      _c2_body,
      [jax.ShapeDtypeStruct((_N, 64), f32), jax.ShapeDtypeStruct((_N, 128), f32)],
      v4p, s, dinv_cat, dinv_uv,
      W_u0, b1r(b_u0), W_v0, b1r(b_v0), W_uv0, b1r(b_uv0))

  tpp = _segsum_sc(g1, src_cat, dst_cat, zn128, 128, _N)

  x2, g2 = _dense(
      _c3_body,
      [jax.ShapeDtypeStruct((_N, 64), f32), jax.ShapeDtypeStruct((_N, 128), f32)],
      tpp, h1, dinv_cat, dinv_uv, W_u1, b1r(b_u1), W_v1, b1r(b_v1))

  t2p = _segsum_sc(g2, src_uv, dst_uv, zn128, 128, _N)

  ufeat, k5 = _dense(
      _c4_body,
      [jax.ShapeDtypeStruct((_U, _FD), f32),
       jax.ShapeDtypeStruct((_R, _FD, _N - _U), f32)],
      t2p, x2, dinv_uv, W_uv1, b1r(b_uv1), W_du, b1r(b_du), W_di, b1r(b_di),
      basis.reshape(_NB, _FD, _FD), coefs)

  qi = jnp.transpose(k5, (1, 2, 0)).reshape(_FD, (_N - _U) * _R)

  ncols = (_N - _U) * _R
  bn = 1280
  out2d = pl.pallas_call(
      _c5_body,
      grid=(ncols // bn,),
      in_specs=[
          pl.BlockSpec((_U, _FD), lambda j: (0, 0)),
          pl.BlockSpec((_FD, bn), lambda j: (0, j)),
      ],
      out_specs=pl.BlockSpec((_U, bn), lambda j: (0, j)),
      out_shape=jax.ShapeDtypeStruct((_U, ncols), f32),
  )(ufeat, qi)

  return out2d.reshape(_U * (_N - _U), _R)
